# baseline (speedup 1.0000x reference)
import functools

import jax
import jax.numpy as jnp
from jax import lax
from jax.experimental import pallas as pl
from jax.experimental.pallas import tpu as pltpu
from jax.experimental.pallas import tpu_sc as plsc

NC = 2
NS = 16
NW = NC * NS

_GATHER_CHUNK = 200
_SCATTER_CHUNK = 200
_EDGE_BLOCK = 6400
_FIN_BLOCK = 2000


def _sc_gather(table, idx_s, idx_t):
    E = idx_s.shape[0]
    D = table.shape[1]
    per_w = E // NW
    C = _GATHER_CHUNK
    n_chunks = per_w // C
    mesh = plsc.VectorSubcoreMesh(core_axis_name="c", subcore_axis_name="s")

    @functools.partial(
        pl.kernel,
        mesh=mesh,
        out_type=[
            jax.ShapeDtypeStruct((E, D), jnp.float32),
            jax.ShapeDtypeStruct((E, D), jnp.float32),
        ],
        scratch_types=[
            pltpu.VMEM((C,), jnp.int32),
            pltpu.VMEM((C,), jnp.int32),
            pltpu.VMEM((C,), jnp.int32),
            pltpu.VMEM((C,), jnp.int32),
            pltpu.VMEM((2, C, D), jnp.float32),
            pltpu.VMEM((2, C, D), jnp.float32),
            pltpu.SemaphoreType.DMA((2,)),
        ],
    )
    def gather_kernel(table_hbm, is_hbm, it_hbm, ni_hbm, nj_hbm,
                      is_v0, is_v1, it_v0, it_v1, rs_v, rt_v, gsem):
        wid = lax.axis_index("s") * NC + lax.axis_index("c")
        base = wid * per_w
        is_bufs = (is_v0, is_v1)
        it_bufs = (it_v0, it_v1)

        def issue(i, b):
            off = base + i * C
            pltpu.sync_copy(is_hbm.at[pl.ds(off, C)], is_bufs[b])
            pltpu.sync_copy(it_hbm.at[pl.ds(off, C)], it_bufs[b])
            pltpu.async_copy(table_hbm.at[is_bufs[b]], rs_v.at[b], gsem.at[b])
            pltpu.async_copy(table_hbm.at[it_bufs[b]], rt_v.at[b], gsem.at[b])

        issue(0, 0)
        n_iter = (n_chunks + 1) // 2 * 2

        @pl.loop(0, n_iter, step=2)
        def _(i):
            for b in (0, 1):
                ii = i + b

                @pl.when(ii + 1 < n_chunks)
                def _():
                    issue(ii + 1, b ^ 1)

                @pl.when(ii < n_chunks)
                def _():
                    pltpu.make_async_copy(table_hbm.at[is_bufs[b]], rs_v.at[b],
                                          gsem.at[b]).wait()
                    pltpu.make_async_copy(table_hbm.at[it_bufs[b]], rt_v.at[b],
                                          gsem.at[b]).wait()
                    off = base + ii * C
                    pltpu.sync_copy(rs_v.at[b], ni_hbm.at[pl.ds(off, C)])
                    pltpu.sync_copy(rt_v.at[b], nj_hbm.at[pl.ds(off, C)])

    return gather_kernel(table, idx_s, idx_t)


def _tc_compute(ni, nj, r3, comb16, pw16, wbig, wsmall, blk_off):
    Es, D = ni.shape
    B = _EDGE_BLOCK
    K1 = comb16.shape[1]
    K2 = pw16.shape[1]
    bf16 = jnp.bfloat16

    def body(ni_ref, nj_ref, r_ref, cb_ref, pw_ref, wb_ref, ws_ref, z_ref):
        ni_b = ni_ref[...]
        nj_b = nj_ref[...]
        inv = 1.0 / r_ref[0, 0, :]
        d_b = (nj_b - ni_b) * inv[:, None]
        fe = jnp.concatenate([ni_b, nj_b, d_b], axis=1).astype(bf16)
        gm = jnp.dot(fe, wb_ref[...], preferred_element_type=jnp.float32)
        gate_pre = gm[:, :D]
        mlp_pre = gm[:, D:]
        g = 1.0 / (1.0 + jnp.exp(-gate_pre))
        m = jnp.where(mlp_pre > 0.0,
                      mlp_pre,
                      jnp.exp(jnp.minimum(mlp_pre, 0.0)) - 1.0)
        sin = jnp.concatenate([cb_ref[...], pw_ref[...]], axis=1)
        sm = jnp.dot(sin, ws_ref[...], preferred_element_type=jnp.float32)
        z1 = sm[:, :D]
        z2 = sm[:, D:2 * D] * (1.0 / (1.0 + jnp.exp(-sm[:, 2 * D:])))
        z_ref[...] = g * m * (z1 + z2)

    return pl.pallas_call(
        body,
        grid=(Es // B,),
        in_specs=[
            pl.BlockSpec((B, D), lambda i: (i, 0)),
            pl.BlockSpec((B, D), lambda i: (i, 0)),
            pl.BlockSpec((1, 1, B), lambda i: (i + blk_off, 0, 0)),
            pl.BlockSpec((B, K1), lambda i: (i + blk_off, 0)),
            pl.BlockSpec((B, K2), lambda i: (i + blk_off, 0)),
            pl.BlockSpec((3 * D, 2 * D), lambda i: (0, 0)),
            pl.BlockSpec((K1 + K2, 3 * D), lambda i: (0, 0)),
        ],
        out_specs=pl.BlockSpec((B, D), lambda i: (i, 0)),
        out_shape=jax.ShapeDtypeStruct((Es, D), jnp.float32),
    )(ni, nj, r3, comb16, pw16, wbig, wsmall)


def _sc_scatter(z, idx_s, table):
    E, D = z.shape
    N = table.shape[0]
    per_w = E // NW
    C = _SCATTER_CHUNK
    n_chunks = per_w // C
    rows_per_sub = (N // NS) // 8 * 8
    tail_rows = N - NS * rows_per_sub
    tail0 = NS * rows_per_sub
    mesh = plsc.VectorSubcoreMesh(core_axis_name="c", subcore_axis_name="s")

    @functools.partial(
        pl.kernel,
        mesh=mesh,
        out_type=jax.ShapeDtypeStruct((NC, N, D), jnp.float32),
        scratch_types=[
            pltpu.VMEM((C,), jnp.int32),
            pltpu.VMEM((C, D), jnp.float32),
            pltpu.VMEM_SHARED((N, D), jnp.float32),
        ],
    )
    def scatter_kernel(z_hbm, is_hbm, tab_hbm, out_hbm, idx_v, z_v, acc):
        cid = lax.axis_index("c")
        sid = lax.axis_index("s")
        row0 = sid * rows_per_sub
        pltpu.sync_copy(tab_hbm.at[pl.ds(row0, rows_per_sub)],
                        acc.at[pl.ds(row0, rows_per_sub)])
        if tail_rows:
            @pl.when(sid == NS - 1)
            def _():
                pltpu.sync_copy(tab_hbm.at[pl.ds(tail0, tail_rows)],
                                acc.at[pl.ds(tail0, tail_rows)])
        plsc.subcore_barrier()

        wid = sid * NC + cid
        base = wid * per_w

        @pl.loop(0, n_chunks)
        def _(i):
            off = base + i * C
            pltpu.sync_copy(is_hbm.at[pl.ds(off, C)], idx_v)
            pltpu.sync_copy(z_hbm.at[pl.ds(off, C)], z_v)
            pltpu.sync_copy(z_v, acc.at[idx_v], add=True)

        plsc.subcore_barrier()
        pltpu.sync_copy(acc.at[pl.ds(row0, rows_per_sub)],
                        out_hbm.at[cid, pl.ds(row0, rows_per_sub)])
        if tail_rows:
            @pl.when(sid == NS - 1)
            def _():
                pltpu.sync_copy(acc.at[pl.ds(tail0, tail_rows)],
                                out_hbm.at[cid, pl.ds(tail0, tail_rows)])

    return scatter_kernel(z, idx_s, table)


def _tc_finalize(inp, parts_list):
    N, D = inp.shape
    B = _FIN_BLOCK
    n_seed = float(len(parts_list) * NC - 1)

    def body(*refs):
        inp_ref = refs[0]
        out_ref = refs[-1]
        acc = -n_seed * inp_ref[...]
        for p_ref in refs[1:-1]:
            acc = acc + p_ref[0] + p_ref[1]
        out_ref[...] = acc

    return pl.pallas_call(
        body,
        grid=(N // B,),
        in_specs=[pl.BlockSpec((B, D), lambda i: (i, 0))] + [
            pl.BlockSpec((NC, B, D), lambda i: (0, i, 0))
            for _ in parts_list
        ],
        out_specs=pl.BlockSpec((B, D), lambda i: (i, 0)),
        out_shape=jax.ShapeDtypeStruct((N, D), jnp.float32),
    )(inp, *parts_list)


_SEGMENT_UNITS = (10, 10, 10, 10, 10)


def kernel(input, nodes, edge_sources, edge_targets, rij, combine_sets,
           plane_wave, W1v, W2v, W2vg, Wg, Wm):
    E = edge_sources.shape[0]
    D = input.shape[1]
    B = _EDGE_BLOCK
    K1 = combine_sets.shape[1]
    K2 = plane_wave.shape[1]
    bf16 = jnp.bfloat16
    assert sum(_SEGMENT_UNITS) * B == E

    r3 = rij.reshape(E // B, 1, B)
    comb16 = combine_sets.astype(bf16)
    pw16 = plane_wave.astype(bf16)
    wbig = jnp.concatenate([Wg, Wm], axis=1).astype(bf16)
    wsmall = jnp.concatenate([
        jnp.concatenate([W1v, jnp.zeros((K1, 2 * D), W1v.dtype)], axis=1),
        jnp.concatenate([jnp.zeros((K2, D), W2v.dtype), W2v, W2vg], axis=1),
    ], axis=0).astype(bf16)

    parts_list = []
    blk_off = 0
    for units in _SEGMENT_UNITS:
        e0 = blk_off * B
        sl = slice(e0, e0 + units * B)
        ni, nj = _sc_gather(input, edge_sources[sl], edge_targets[sl])
        z = _tc_compute(ni, nj, r3, comb16, pw16, wbig, wsmall, blk_off)
        parts_list.append(_sc_scatter(z, edge_sources[sl], input))
        blk_off += units
    return _tc_finalize(input, parts_list)


# speedup vs baseline: 1.0094x; 1.0094x over previous
import functools

import jax
import jax.numpy as jnp
from jax import lax
from jax.experimental import pallas as pl
from jax.experimental.pallas import tpu as pltpu
from jax.experimental.pallas import tpu_sc as plsc

NC = 2
NS = 16
NW = NC * NS

_GATHER_CHUNK = 200
_SCATTER_CHUNK = 200
_EDGE_BLOCK = 6400
_FIN_BLOCK = 2000


def _sc_gather(table, idx_s, idx_t):
    E = idx_s.shape[0]
    D = table.shape[1]
    per_w = E // NW
    C = _GATHER_CHUNK
    n_chunks = per_w // C
    mesh = plsc.VectorSubcoreMesh(core_axis_name="c", subcore_axis_name="s")

    @functools.partial(
        pl.kernel,
        mesh=mesh,
        out_type=[
            jax.ShapeDtypeStruct((E, D), jnp.float32),
            jax.ShapeDtypeStruct((E, D), jnp.float32),
        ],
        scratch_types=[
            pltpu.VMEM((C,), jnp.int32),
            pltpu.VMEM((C,), jnp.int32),
            pltpu.VMEM((C,), jnp.int32),
            pltpu.VMEM((C,), jnp.int32),
            pltpu.VMEM((2, C, D), jnp.float32),
            pltpu.VMEM((2, C, D), jnp.float32),
            pltpu.SemaphoreType.DMA((2,)),
        ],
    )
    def gather_kernel(table_hbm, is_hbm, it_hbm, ni_hbm, nj_hbm,
                      is_v0, is_v1, it_v0, it_v1, rs_v, rt_v, gsem):
        wid = lax.axis_index("s") * NC + lax.axis_index("c")
        base = wid * per_w
        is_bufs = (is_v0, is_v1)
        it_bufs = (it_v0, it_v1)

        def issue(i, b):
            off = base + i * C
            pltpu.sync_copy(is_hbm.at[pl.ds(off, C)], is_bufs[b])
            pltpu.sync_copy(it_hbm.at[pl.ds(off, C)], it_bufs[b])
            pltpu.async_copy(table_hbm.at[is_bufs[b]], rs_v.at[b], gsem.at[b])
            pltpu.async_copy(table_hbm.at[it_bufs[b]], rt_v.at[b], gsem.at[b])

        issue(0, 0)
        n_iter = (n_chunks + 1) // 2 * 2

        @pl.loop(0, n_iter, step=2)
        def _(i):
            for b in (0, 1):
                ii = i + b

                @pl.when(ii + 1 < n_chunks)
                def _():
                    issue(ii + 1, b ^ 1)

                @pl.when(ii < n_chunks)
                def _():
                    pltpu.make_async_copy(table_hbm.at[is_bufs[b]], rs_v.at[b],
                                          gsem.at[b]).wait()
                    pltpu.make_async_copy(table_hbm.at[it_bufs[b]], rt_v.at[b],
                                          gsem.at[b]).wait()
                    off = base + ii * C
                    pltpu.sync_copy(rs_v.at[b], ni_hbm.at[pl.ds(off, C)])
                    pltpu.sync_copy(rt_v.at[b], nj_hbm.at[pl.ds(off, C)])

    return gather_kernel(table, idx_s, idx_t)


def _tc_compute(ni, nj, r3, comb16, pw16, wbig, wsmall, blk_off):
    Es, D = ni.shape
    B = _EDGE_BLOCK
    K1 = comb16.shape[1]
    K2 = pw16.shape[1]
    bf16 = jnp.bfloat16

    def body(ni_ref, nj_ref, r_ref, cb_ref, pw_ref, wb_ref, ws_ref, z_ref):
        ni_b = ni_ref[...]
        nj_b = nj_ref[...]
        inv = 1.0 / r_ref[0, 0, :]
        d_b = (nj_b - ni_b) * inv[:, None]
        fe = jnp.concatenate([ni_b, nj_b, d_b], axis=1).astype(bf16)
        gm = jnp.dot(fe, wb_ref[...], preferred_element_type=jnp.float32)
        gate_pre = gm[:, :D]
        mlp_pre = gm[:, D:]
        g = 1.0 / (1.0 + jnp.exp(-gate_pre))
        m = jnp.where(mlp_pre > 0.0,
                      mlp_pre,
                      jnp.exp(jnp.minimum(mlp_pre, 0.0)) - 1.0)
        sin = jnp.concatenate([cb_ref[...], pw_ref[...]], axis=1)
        sm = jnp.dot(sin, ws_ref[...], preferred_element_type=jnp.float32)
        z1 = sm[:, :D]
        z2 = sm[:, D:2 * D] * (1.0 / (1.0 + jnp.exp(-sm[:, 2 * D:])))
        z_ref[...] = g * m * (z1 + z2)

    return pl.pallas_call(
        body,
        grid=(Es // B,),
        in_specs=[
            pl.BlockSpec((B, D), lambda i: (i, 0)),
            pl.BlockSpec((B, D), lambda i: (i, 0)),
            pl.BlockSpec((1, 1, B), lambda i: (i + blk_off, 0, 0)),
            pl.BlockSpec((B, K1), lambda i: (i + blk_off, 0)),
            pl.BlockSpec((B, K2), lambda i: (i + blk_off, 0)),
            pl.BlockSpec((3 * D, 2 * D), lambda i: (0, 0)),
            pl.BlockSpec((K1 + K2, 3 * D), lambda i: (0, 0)),
        ],
        out_specs=pl.BlockSpec((B, D), lambda i: (i, 0)),
        out_shape=jax.ShapeDtypeStruct((Es, D), jnp.float32),
    )(ni, nj, r3, comb16, pw16, wbig, wsmall)


def _sc_scatter(z, idx_s, table):
    E, D = z.shape
    N = table.shape[0]
    per_w = E // NW
    C = _SCATTER_CHUNK
    n_chunks = per_w // C
    rows_per_sub = (N // NS) // 8 * 8
    tail_rows = N - NS * rows_per_sub
    tail0 = NS * rows_per_sub
    mesh = plsc.VectorSubcoreMesh(core_axis_name="c", subcore_axis_name="s")

    @functools.partial(
        pl.kernel,
        mesh=mesh,
        out_type=jax.ShapeDtypeStruct((NC, N, D), jnp.float32),
        scratch_types=[
            pltpu.VMEM((C,), jnp.int32),
            pltpu.VMEM((C, D), jnp.float32),
            pltpu.VMEM_SHARED((N, D), jnp.float32),
        ],
    )
    def scatter_kernel(z_hbm, is_hbm, tab_hbm, out_hbm, idx_v, z_v, acc):
        cid = lax.axis_index("c")
        sid = lax.axis_index("s")
        row0 = sid * rows_per_sub
        pltpu.sync_copy(tab_hbm.at[pl.ds(row0, rows_per_sub)],
                        acc.at[pl.ds(row0, rows_per_sub)])
        if tail_rows:
            @pl.when(sid == NS - 1)
            def _():
                pltpu.sync_copy(tab_hbm.at[pl.ds(tail0, tail_rows)],
                                acc.at[pl.ds(tail0, tail_rows)])
        plsc.subcore_barrier()

        wid = sid * NC + cid
        base = wid * per_w

        @pl.loop(0, n_chunks)
        def _(i):
            off = base + i * C
            pltpu.sync_copy(is_hbm.at[pl.ds(off, C)], idx_v)
            pltpu.sync_copy(z_hbm.at[pl.ds(off, C)], z_v)
            pltpu.sync_copy(z_v, acc.at[idx_v], add=True)

        plsc.subcore_barrier()
        pltpu.sync_copy(acc.at[pl.ds(row0, rows_per_sub)],
                        out_hbm.at[cid, pl.ds(row0, rows_per_sub)])
        if tail_rows:
            @pl.when(sid == NS - 1)
            def _():
                pltpu.sync_copy(acc.at[pl.ds(tail0, tail_rows)],
                                out_hbm.at[cid, pl.ds(tail0, tail_rows)])

    return scatter_kernel(z, idx_s, table)


def _tc_finalize(inp, parts_list):
    N, D = inp.shape
    B = _FIN_BLOCK
    n_seed = float(len(parts_list) * NC - 1)

    def body(*refs):
        inp_ref = refs[0]
        out_ref = refs[-1]
        acc = -n_seed * inp_ref[...]
        for p_ref in refs[1:-1]:
            acc = acc + p_ref[0] + p_ref[1]
        out_ref[...] = acc

    return pl.pallas_call(
        body,
        grid=(N // B,),
        in_specs=[pl.BlockSpec((B, D), lambda i: (i, 0))] + [
            pl.BlockSpec((NC, B, D), lambda i: (0, i, 0))
            for _ in parts_list
        ],
        out_specs=pl.BlockSpec((B, D), lambda i: (i, 0)),
        out_shape=jax.ShapeDtypeStruct((N, D), jnp.float32),
    )(inp, *parts_list)


_SEGMENT_UNITS = (14, 14, 12, 10)


def kernel(input, nodes, edge_sources, edge_targets, rij, combine_sets,
           plane_wave, W1v, W2v, W2vg, Wg, Wm):
    E = edge_sources.shape[0]
    D = input.shape[1]
    B = _EDGE_BLOCK
    K1 = combine_sets.shape[1]
    K2 = plane_wave.shape[1]
    bf16 = jnp.bfloat16
    assert sum(_SEGMENT_UNITS) * B == E

    r3 = rij.reshape(E // B, 1, B)
    comb16 = combine_sets.astype(bf16)
    pw16 = plane_wave.astype(bf16)
    wbig = jnp.concatenate([Wg, Wm], axis=1).astype(bf16)
    wsmall = jnp.concatenate([
        jnp.concatenate([W1v, jnp.zeros((K1, 2 * D), W1v.dtype)], axis=1),
        jnp.concatenate([jnp.zeros((K2, D), W2v.dtype), W2v, W2vg], axis=1),
    ], axis=0).astype(bf16)

    parts_list = []
    blk_off = 0
    for units in _SEGMENT_UNITS:
        e0 = blk_off * B
        sl = slice(e0, e0 + units * B)
        ni, nj = _sc_gather(input, edge_sources[sl], edge_targets[sl])
        z = _tc_compute(ni, nj, r3, comb16, pw16, wbig, wsmall, blk_off)
        parts_list.append(_sc_scatter(z, edge_sources[sl], input))
        blk_off += units
    return _tc_finalize(input, parts_list)


# speedup vs baseline: 1.0187x; 1.0092x over previous
import functools

import jax
import jax.numpy as jnp
from jax import lax
from jax.experimental import pallas as pl
from jax.experimental.pallas import tpu as pltpu
from jax.experimental.pallas import tpu_sc as plsc

NC = 2
NS = 16
NW = NC * NS

_GATHER_CHUNK = 200
_SCATTER_CHUNK = 200
_EDGE_BLOCK = 6400
_FIN_BLOCK = 2000


def _sc_gather(table, idx_s, idx_t):
    E = idx_s.shape[0]
    D = table.shape[1]
    per_w = E // NW
    C = _GATHER_CHUNK
    n_chunks = per_w // C
    mesh = plsc.VectorSubcoreMesh(core_axis_name="c", subcore_axis_name="s")

    @functools.partial(
        pl.kernel,
        mesh=mesh,
        out_type=[
            jax.ShapeDtypeStruct((E, D), jnp.float32),
            jax.ShapeDtypeStruct((E, D), jnp.float32),
        ],
        scratch_types=[
            pltpu.VMEM((C,), jnp.int32),
            pltpu.VMEM((C,), jnp.int32),
            pltpu.VMEM((C,), jnp.int32),
            pltpu.VMEM((C,), jnp.int32),
            pltpu.VMEM((2, C, D), jnp.float32),
            pltpu.VMEM((2, C, D), jnp.float32),
            pltpu.SemaphoreType.DMA((2,)),
        ],
    )
    def gather_kernel(table_hbm, is_hbm, it_hbm, ni_hbm, nj_hbm,
                      is_v0, is_v1, it_v0, it_v1, rs_v, rt_v, gsem):
        wid = lax.axis_index("s") * NC + lax.axis_index("c")
        base = wid * per_w
        is_bufs = (is_v0, is_v1)
        it_bufs = (it_v0, it_v1)

        def issue(i, b):
            off = base + i * C
            pltpu.sync_copy(is_hbm.at[pl.ds(off, C)], is_bufs[b])
            pltpu.sync_copy(it_hbm.at[pl.ds(off, C)], it_bufs[b])
            pltpu.async_copy(table_hbm.at[is_bufs[b]], rs_v.at[b], gsem.at[b])
            pltpu.async_copy(table_hbm.at[it_bufs[b]], rt_v.at[b], gsem.at[b])

        issue(0, 0)
        n_iter = (n_chunks + 1) // 2 * 2

        @pl.loop(0, n_iter, step=2)
        def _(i):
            for b in (0, 1):
                ii = i + b

                @pl.when(ii + 1 < n_chunks)
                def _():
                    issue(ii + 1, b ^ 1)

                @pl.when(ii < n_chunks)
                def _():
                    pltpu.make_async_copy(table_hbm.at[is_bufs[b]], rs_v.at[b],
                                          gsem.at[b]).wait()
                    pltpu.make_async_copy(table_hbm.at[it_bufs[b]], rt_v.at[b],
                                          gsem.at[b]).wait()
                    off = base + ii * C
                    pltpu.sync_copy(rs_v.at[b], ni_hbm.at[pl.ds(off, C)])
                    pltpu.sync_copy(rt_v.at[b], nj_hbm.at[pl.ds(off, C)])

    return gather_kernel(table, idx_s, idx_t)


def _tc_compute(ni, nj, r3, comb16, pw16, wbig, wsmall, blk_off):
    Es, D = ni.shape
    B = _EDGE_BLOCK
    K1 = comb16.shape[1]
    K2 = pw16.shape[1]
    bf16 = jnp.bfloat16

    def body(ni_ref, nj_ref, r_ref, cb_ref, pw_ref, wb_ref, ws_ref, z_ref):
        ni_b = ni_ref[...]
        nj_b = nj_ref[...]
        inv = 1.0 / r_ref[0, 0, :]
        d_b = (nj_b - ni_b) * inv[:, None]
        fe = jnp.concatenate([ni_b, nj_b, d_b], axis=1).astype(bf16)
        gm = jnp.dot(fe, wb_ref[...], preferred_element_type=jnp.float32)
        gate_pre = gm[:, :D]
        mlp_pre = gm[:, D:]
        g = 1.0 / (1.0 + jnp.exp(-gate_pre))
        m = jnp.where(mlp_pre > 0.0,
                      mlp_pre,
                      jnp.exp(jnp.minimum(mlp_pre, 0.0)) - 1.0)
        sin = jnp.concatenate([cb_ref[...], pw_ref[...]], axis=1)
        sm = jnp.dot(sin, ws_ref[...], preferred_element_type=jnp.float32)
        z1 = sm[:, :D]
        z2 = sm[:, D:2 * D] * (1.0 / (1.0 + jnp.exp(-sm[:, 2 * D:])))
        z_ref[...] = g * m * (z1 + z2)

    return pl.pallas_call(
        body,
        grid=(Es // B,),
        in_specs=[
            pl.BlockSpec((B, D), lambda i: (i, 0)),
            pl.BlockSpec((B, D), lambda i: (i, 0)),
            pl.BlockSpec((1, 1, B), lambda i: (i + blk_off, 0, 0)),
            pl.BlockSpec((B, K1), lambda i: (i + blk_off, 0)),
            pl.BlockSpec((B, K2), lambda i: (i + blk_off, 0)),
            pl.BlockSpec((3 * D, 2 * D), lambda i: (0, 0)),
            pl.BlockSpec((K1 + K2, 3 * D), lambda i: (0, 0)),
        ],
        out_specs=pl.BlockSpec((B, D), lambda i: (i, 0)),
        out_shape=jax.ShapeDtypeStruct((Es, D), jnp.float32),
    )(ni, nj, r3, comb16, pw16, wbig, wsmall)


def _sc_scatter(z, idx_s, table):
    E, D = z.shape
    N = table.shape[0]
    per_w = E // NW
    C = _SCATTER_CHUNK
    n_chunks = per_w // C
    rows_per_sub = (N // NS) // 8 * 8
    tail_rows = N - NS * rows_per_sub
    tail0 = NS * rows_per_sub
    mesh = plsc.VectorSubcoreMesh(core_axis_name="c", subcore_axis_name="s")

    @functools.partial(
        pl.kernel,
        mesh=mesh,
        out_type=jax.ShapeDtypeStruct((NC, N, D), jnp.float32),
        scratch_types=[
            pltpu.VMEM((C,), jnp.int32),
            pltpu.VMEM((C, D), jnp.float32),
            pltpu.VMEM_SHARED((N, D), jnp.float32),
        ],
    )
    def scatter_kernel(z_hbm, is_hbm, tab_hbm, out_hbm, idx_v, z_v, acc):
        cid = lax.axis_index("c")
        sid = lax.axis_index("s")
        row0 = sid * rows_per_sub
        pltpu.sync_copy(tab_hbm.at[pl.ds(row0, rows_per_sub)],
                        acc.at[pl.ds(row0, rows_per_sub)])
        if tail_rows:
            @pl.when(sid == NS - 1)
            def _():
                pltpu.sync_copy(tab_hbm.at[pl.ds(tail0, tail_rows)],
                                acc.at[pl.ds(tail0, tail_rows)])
        plsc.subcore_barrier()

        wid = sid * NC + cid
        base = wid * per_w

        @pl.loop(0, n_chunks)
        def _(i):
            off = base + i * C
            pltpu.sync_copy(is_hbm.at[pl.ds(off, C)], idx_v)
            pltpu.sync_copy(z_hbm.at[pl.ds(off, C)], z_v)
            pltpu.sync_copy(z_v, acc.at[idx_v], add=True)

        plsc.subcore_barrier()
        pltpu.sync_copy(acc.at[pl.ds(row0, rows_per_sub)],
                        out_hbm.at[cid, pl.ds(row0, rows_per_sub)])
        if tail_rows:
            @pl.when(sid == NS - 1)
            def _():
                pltpu.sync_copy(acc.at[pl.ds(tail0, tail_rows)],
                                out_hbm.at[cid, pl.ds(tail0, tail_rows)])

    return scatter_kernel(z, idx_s, table)


def _tc_finalize(inp, parts_list):
    N, D = inp.shape
    B = _FIN_BLOCK
    n_seed = float(len(parts_list) * NC - 1)

    def body(*refs):
        inp_ref = refs[0]
        out_ref = refs[-1]
        acc = -n_seed * inp_ref[...]
        for p_ref in refs[1:-1]:
            acc = acc + p_ref[0] + p_ref[1]
        out_ref[...] = acc

    return pl.pallas_call(
        body,
        grid=(N // B,),
        in_specs=[pl.BlockSpec((B, D), lambda i: (i, 0))] + [
            pl.BlockSpec((NC, B, D), lambda i: (0, i, 0))
            for _ in parts_list
        ],
        out_specs=pl.BlockSpec((B, D), lambda i: (i, 0)),
        out_shape=jax.ShapeDtypeStruct((N, D), jnp.float32),
    )(inp, *parts_list)


_SEGMENT_UNITS = (13, 13, 12, 12)


def kernel(input, nodes, edge_sources, edge_targets, rij, combine_sets,
           plane_wave, W1v, W2v, W2vg, Wg, Wm):
    E = edge_sources.shape[0]
    D = input.shape[1]
    B = _EDGE_BLOCK
    K1 = combine_sets.shape[1]
    K2 = plane_wave.shape[1]
    bf16 = jnp.bfloat16
    assert sum(_SEGMENT_UNITS) * B == E

    r3 = rij.reshape(E // B, 1, B)
    comb16 = combine_sets.astype(bf16)
    pw16 = plane_wave.astype(bf16)
    wbig = jnp.concatenate([Wg, Wm], axis=1).astype(bf16)
    wsmall = jnp.concatenate([
        jnp.concatenate([W1v, jnp.zeros((K1, 2 * D), W1v.dtype)], axis=1),
        jnp.concatenate([jnp.zeros((K2, D), W2v.dtype), W2v, W2vg], axis=1),
    ], axis=0).astype(bf16)

    parts_list = []
    blk_off = 0
    for units in _SEGMENT_UNITS:
        e0 = blk_off * B
        sl = slice(e0, e0 + units * B)
        ni, nj = _sc_gather(input, edge_sources[sl], edge_targets[sl])
        z = _tc_compute(ni, nj, r3, comb16, pw16, wbig, wsmall, blk_off)
        parts_list.append(_sc_scatter(z, edge_sources[sl], input))
        blk_off += units
    return _tc_finalize(input, parts_list)
